# baseline (speedup 1.0000x reference)
import functools

import jax
import jax.numpy as jnp
from jax import lax
from jax.experimental import pallas as pl
from jax.experimental.pallas import tpu as pltpu
from jax.experimental.pallas import tpu_sc as plsc

_BATCH = 16384
_D = 64
_LANES = 16
_ROW = 128
_DUMP = _BATCH
_SCR = _BATCH + 8

_info = plsc.get_sparse_core_info()
_NC, _NS = _info.num_cores, _info.num_subcores
_NW = _NC * _NS
_BPW = _BATCH // _NW
_PIECE = 512
_NGRP = _BATCH // _LANES
_ALIST = _BATCH + 8 * _NGRP + _LANES


def _extract_body(n, nbfull, base_blocks, rem, pmax, tail_w,
                  idx_hbm, tab_hbm, tail_hbm, scr_hbm,
                  all_idx, au, ap, pbuf, ptail, growbuf, posb,
                  out_sem, sem):
    wid = lax.axis_index("s") * _NC + lax.axis_index("c")
    first_blk = wid * base_blocks + jnp.minimum(wid, rem)
    nxt_blk = (wid + 1) * base_blocks + jnp.minimum(wid + 1, rem)
    slab_lo = first_blk * 128
    slab_hi = jnp.minimum(nxt_blk * 128, n)

    pltpu.sync_copy(idx_hbm, all_idx)

    iota = lax.iota(jnp.int32, _LANES)

    sent = jnp.full((_LANES,), jnp.int32(0x7FFFFFFF))

    def compact(g, cnt):
        v = all_idx[pl.ds(g * _LANES, _LANES)]
        m = (v >= slab_lo) & (v < slab_hi)
        slot = (cnt + 7) & ~7
        au[pl.ds(slot, _LANES)] = sent
        plsc.store_compressed(au.at[pl.ds(slot, _LANES)], v, mask=m)
        p = g * _LANES + iota
        plsc.store_compressed(ap.at[pl.ds(slot, _LANES)], p, mask=m)
        npos = plsc.all_reduce_population_count(m)
        return slot + npos[0]

    cnt = lax.fori_loop(0, _NGRP, compact, jnp.int32(0))
    ngroups = (cnt + _LANES - 1) // _LANES

    def drain_one():
        pltpu.make_async_copy(growbuf.at[0], scr_hbm.at[pl.ds(0, _LANES)],
                              out_sem).wait()

    def process_piece(buf, lo, width, h0):
        def scan(g, h):
            v = au[pl.ds(g * _LANES, _LANES)]
            m = (v >= lo) & (v < lo + width)
            nhit = plsc.all_reduce_population_count(m)
            r = h % 4

            @pl.when(nhit[0] > 0)
            def _do():
                @pl.when(h >= 4)
                def _recycle():
                    drain_one()
                pos = ap[pl.ds(g * _LANES, _LANES)]
                posb[r, :] = jnp.where(m, pos, jnp.int32(_DUMP))
                for j in range(_LANES):
                    vj = v[j]

                    @pl.when((vj >= lo) & (vj < lo + width))
                    def _one(j=j, vj=vj):
                        c = vj - lo
                        cv = jnp.full((_LANES,), c, jnp.int32)
                        for k in range(_D // _LANES):
                            dv = k * _LANES + iota
                            val = plsc.load_gather(buf, [dv, cv])
                            growbuf[r, j, pl.ds(k * _LANES, _LANES)] = val
                pltpu.async_copy(growbuf.at[r], scr_hbm.at[posb.at[r]],
                                 out_sem)
            return jnp.where(nhit[0] > 0, h + 1, h)

        return lax.fori_loop(0, ngroups, scan, h0)

    def piece(p, h):
        nominal = (first_blk + p * (_PIECE // 128)) * 128
        start = jnp.minimum(nominal, (nbfull - _PIECE // 128) * 128)
        start = pl.multiple_of(start, 128)
        pltpu.sync_copy(tab_hbm.at[:, pl.ds(start, _PIECE)], pbuf)
        return process_piece(pbuf, start, jnp.int32(_PIECE), h)

    h = lax.fori_loop(0, pmax, piece, jnp.int32(0))

    if tail_w:
        pltpu.sync_copy(tail_hbm, ptail)
        h = process_piece(ptail, jnp.int32(nbfull * 128), jnp.int32(tail_w),
                          h)

    for k in range(4):
        @pl.when(h >= k + 1)
        def _final_drain():
            drain_one()


def _dot_body(scru_hbm, scri_hbm, out_hbm, ubuf, ibuf, out_v, sem):
    wid = lax.axis_index("s") * _NC + lax.axis_index("c")
    base = wid * _BPW
    iota = lax.iota(jnp.int32, _LANES)

    for c in range(_BPW // 128):
        pltpu.sync_copy(scru_hbm.at[pl.ds(base + c * 128, 128)], ubuf)
        pltpu.sync_copy(scri_hbm.at[pl.ds(base + c * 128, 128)], ibuf)

        def group(k, carry, c=c):
            row = k * _LANES + iota
            acc = jnp.zeros((_LANES,), jnp.float32)
            for d in range(_D):
                dv = jnp.full((_LANES,), d, jnp.int32)
                u = plsc.load_gather(ubuf, [row, dv])
                v = plsc.load_gather(ibuf, [row, dv])
                acc = acc + u * v
            out_v[pl.ds(c * 128 + k * _LANES, _LANES)] = acc
            return carry

        lax.fori_loop(0, 128 // _LANES, group, 0)

    pltpu.sync_copy(out_v, out_hbm.at[pl.ds(base, _BPW)])


def _make_extract(n):
    nbfull = n // 128
    tail_w = n - nbfull * 128
    nblocks = nbfull + (1 if tail_w else 0)
    base_blocks, rem = nblocks // _NW, nblocks % _NW
    pmax = -(-(base_blocks + 1) * 128 // _PIECE)
    mesh = plsc.VectorSubcoreMesh(core_axis_name="c", subcore_axis_name="s")
    return pl.kernel(
        functools.partial(_extract_body, n, nbfull, base_blocks, rem,
                          pmax, tail_w),
        mesh=mesh,
        out_type=jax.ShapeDtypeStruct((_SCR, _ROW), jnp.float32),
        scratch_types=[
            pltpu.VMEM((_BATCH,), jnp.int32),
            pltpu.VMEM((_ALIST,), jnp.int32),
            pltpu.VMEM((_ALIST,), jnp.int32),
            pltpu.VMEM((_D, _PIECE), jnp.float32),
            pltpu.VMEM((_D, tail_w or 128), jnp.float32),
            pltpu.VMEM((4, _LANES, _ROW), jnp.float32),
            pltpu.VMEM((4, _LANES), jnp.int32),
            pltpu.SemaphoreType.DMA,
            pltpu.SemaphoreType.DMA,
        ],
        compiler_params=pltpu.CompilerParams(needs_layout_passes=False),
    )


def _make_dot():
    mesh = plsc.VectorSubcoreMesh(core_axis_name="c", subcore_axis_name="s")
    return pl.kernel(
        _dot_body,
        mesh=mesh,
        out_type=jax.ShapeDtypeStruct((_BATCH,), jnp.float32),
        scratch_types=[
            pltpu.VMEM((128, _ROW), jnp.float32),
            pltpu.VMEM((128, _ROW), jnp.float32),
            pltpu.VMEM((_BPW,), jnp.float32),
            pltpu.SemaphoreType.DMA,
        ],
        compiler_params=pltpu.CompilerParams(needs_layout_passes=False),
    )


@jax.jit
def _run(user, item, user_factors, item_factors):
    uft = user_factors.T
    ift = item_factors.T
    tail_u = uft[:, (uft.shape[1] // 128) * 128:]
    tail_i = ift[:, (ift.shape[1] // 128) * 128:]
    scr_u = _make_extract(uft.shape[1])(user, uft, tail_u)
    scr_i = _make_extract(ift.shape[1])(item, ift, tail_i)
    return _make_dot()(scr_u, scr_i)


def kernel(user, item, user_factors, item_factors):
    return _run(user, item, user_factors, item_factors)


# speedup vs baseline: 25.9449x; 25.9449x over previous
import jax
import jax.numpy as jnp
from jax import lax
from jax.experimental import pallas as pl
from jax.experimental.pallas import tpu as pltpu
from jax.experimental.pallas import tpu_sc as plsc

_BATCH = 16384
_D = 64
_W = 2 * _D
_LANES = 16

_info = plsc.get_sparse_core_info()
_NC, _NS = _info.num_cores, _info.num_subcores
_NW = _NC * _NS
_BPW = _BATCH // _NW
_CHUNK = 128
_NCHUNK = _BPW // _CHUNK
_GPC = _CHUNK // _LANES


def _body(user_hbm, item_hbm, uf_hbm, if_hbm, out_hbm,
          idx_ou, idx_oi, idx_du, idx_di, rows_u, rows_i, out_v, sem):
    wid = lax.axis_index("s") * _NC + lax.axis_index("c")
    base = wid * _BPW

    for c in range(_NCHUNK):
        pltpu.sync_copy(user_hbm.at[pl.ds(base + c * _CHUNK, _CHUNK)],
                        idx_ou.at[c])
        pltpu.sync_copy(item_hbm.at[pl.ds(base + c * _CHUNK, _CHUNK)],
                        idx_oi.at[c])

    for c in range(_NCHUNK):
        def shift(k, carry, c=c):
            s = pl.ds(k * _LANES, _LANES)
            idx_du[c, s] = lax.shift_right_logical(idx_ou[c, s], 1)
            idx_di[c, s] = lax.shift_right_logical(idx_oi[c, s], 1)
            return carry
        lax.fori_loop(0, _GPC, shift, 0)

    def fire(c, buf):
        return (
            pltpu.async_copy(uf_hbm.at[idx_du.at[c]], rows_u.at[buf], sem),
            pltpu.async_copy(if_hbm.at[idx_di.at[c]], rows_i.at[buf], sem),
        )

    iota = lax.iota(jnp.int32, _LANES)
    pending = fire(0, 0)
    for c in range(_NCHUNK):
        nxt = fire(c + 1, (c + 1) % 2) if c + 1 < _NCHUNK else None
        pending[0].wait()
        pending[1].wait()
        pending = nxt
        ru = rows_u.at[c % 2]
        ri = rows_i.at[c % 2]

        def group(k, carry, c=c, ru=ru, ri=ri):
            s = pl.ds(k * _LANES, _LANES)
            row = k * _LANES + iota
            col_u = (idx_ou[c, s] & 1) * _D
            col_i = (idx_oi[c, s] & 1) * _D
            acc = jnp.zeros((_LANES,), jnp.float32)
            for d in range(_D):
                u = plsc.load_gather(ru, [row, col_u + d])
                v = plsc.load_gather(ri, [row, col_i + d])
                acc = acc + u * v
            out_v[pl.ds(c * _CHUNK + k * _LANES, _LANES)] = acc
            return carry

        lax.fori_loop(0, _GPC, group, 0)

    pltpu.sync_copy(out_v, out_hbm.at[pl.ds(base, _BPW)])


@jax.jit
def _run(user, item, user_factors, item_factors):
    uf2 = user_factors.reshape(-1, _W)
    if2 = item_factors.reshape(-1, _W)
    mesh = plsc.VectorSubcoreMesh(core_axis_name="c", subcore_axis_name="s")
    fn = pl.kernel(
        _body,
        mesh=mesh,
        out_type=jax.ShapeDtypeStruct((_BATCH,), jnp.float32),
        scratch_types=[
            pltpu.VMEM((_NCHUNK, _CHUNK), jnp.int32),
            pltpu.VMEM((_NCHUNK, _CHUNK), jnp.int32),
            pltpu.VMEM((_NCHUNK, _CHUNK), jnp.int32),
            pltpu.VMEM((_NCHUNK, _CHUNK), jnp.int32),
            pltpu.VMEM((2, _CHUNK, _W), jnp.float32),
            pltpu.VMEM((2, _CHUNK, _W), jnp.float32),
            pltpu.VMEM((_BPW,), jnp.float32),
            pltpu.SemaphoreType.DMA,
        ],
        compiler_params=pltpu.CompilerParams(needs_layout_passes=False),
    )
    return fn(user, item, uf2, if2)


def kernel(user, item, user_factors, item_factors):
    return _run(user, item, user_factors, item_factors)
